# Initial kernel scaffold; baseline (speedup 1.0000x reference)
import jax
import jax.numpy as jnp
from jax.experimental import pallas as pl


def kernel(infeat, labelTpesudo, labelT, FeatureDA):
    raise NotImplementedError("write your pallas kernel here")


# speedup vs baseline: 13.5022x; 13.5022x over previous
import jax
import jax.numpy as jnp
from jax.experimental import pallas as pl
from jax.experimental.pallas import tpu as pltpu

_POOL = 225
_K = 32
_HALF = 16
_NEG = -1e30


def _score_select_kernel(infeat_ref, idx_ref, val_ref):
    x0 = infeat_ref[0, 0]
    x1 = infeat_ref[0, 1]
    m = jnp.maximum(x0, x1)
    e0 = jnp.exp(x0 - m)
    e1 = jnp.exp(x1 - m)
    inv_s = 1.0 / (e0 + e1)
    p0 = e0 * inv_s
    p1 = e1 * inv_s
    ent = -(p0 * jnp.log(p0 + 1e-5) + p1 * jnp.log(p1 + 1e-5))

    rows = jax.lax.broadcasted_iota(jnp.int32, (256, 256), 0)
    cols = jax.lax.broadcasted_iota(jnp.int32, (256, 256), 1)
    band = ((rows >= cols) & (rows < cols + _K) & (cols < _POOL)).astype(
        jnp.float32)
    valid = (rows < _POOL) & (cols < _POOL)
    flat = rows * _POOL + cols

    idxs = []
    vals = []
    for c in range(2):
        g = (p0 if c == 0 else p1) - 0.1 * ent
        tmp = jax.lax.dot_general(
            g, band, (((1,), (0,)), ((), ())),
            preferred_element_type=jnp.float32,
            precision=jax.lax.Precision.HIGHEST)
        score = jax.lax.dot_general(
            band, tmp, (((0,), (0,)), ((), ())),
            preferred_element_type=jnp.float32,
            precision=jax.lax.Precision.HIGHEST)
        score = jnp.where(valid, score * (1.0 / (_K * _K)), _NEG)
        for kk in range(2):
            v = jnp.max(score)
            idx = jnp.min(jnp.where(score == v, flat, jnp.int32(2**31 - 1)))
            py = idx // _POOL
            px = idx - py * _POOL
            idxs.append(idx)
            vals.append(v)
            in_box = ((rows >= py - _HALF) & (rows < py + _HALF) &
                      (cols >= px - _HALF) & (cols < px + _HALF) & valid)
            score = jnp.where(in_box, 0.0, score)
    idx_ref[...] = jnp.stack(idxs).reshape(1, 1, 4)
    val_ref[...] = jnp.stack(vals).reshape(1, 1, 4)


def _extract(x, dy, px):
    sh_y = jax.lax.rem(40 - dy, 40)
    sh_x = jax.lax.rem(256 - px, 256)
    xr = pltpu.roll(x, sh_y, axis=1)[:, :_K, :]
    xc = pltpu.roll(xr, sh_x, axis=2)[:, :, :_K]
    return xc


def _gather_kernel(pref_ref, infeat_hbm, lps_hbm, lt_hbm, fda_hbm,
                   cls_out, fda_out, lt_out, lps_out,
                   s_if, s_fda, s_lt, s_lps,
                   sem0, sem1, sem2, sem3):
    n = pl.program_id(0)
    b = pref_ref[n, 0]
    py = pref_ref[n, 1]
    px = pref_ref[n, 2]
    py0 = (py // 8) * 8
    dy = py - py0
    c0 = pltpu.make_async_copy(
        infeat_hbm.at[b, :, pl.ds(py0, 40), :], s_if, sem0)
    c1 = pltpu.make_async_copy(
        fda_hbm.at[b, :, pl.ds(py0, 40), :], s_fda, sem1)
    c2 = pltpu.make_async_copy(
        lt_hbm.at[b, :, pl.ds(py0, 40), :], s_lt, sem2)
    c3 = pltpu.make_async_copy(
        lps_hbm.at[b, :, pl.ds(py0, 40), :], s_lps, sem3)
    c0.start()
    c1.start()
    c2.start()
    c3.start()
    c0.wait()
    cls_out[0] = _extract(s_if[...], dy, px)
    c1.wait()
    fda_out[0] = _extract(s_fda[...], dy, px)
    c2.wait()
    lt_out[0] = _extract(s_lt[...], dy, px)
    c3.wait()
    lps_out[0] = _extract(s_lps[...], dy, px)


def kernel(infeat, labelTpesudo, labelT, FeatureDA):
    idx8, vals8 = pl.pallas_call(
        _score_select_kernel,
        grid=(8,),
        in_specs=[pl.BlockSpec((1, 2, 256, 256), lambda b: (b, 0, 0, 0))],
        out_specs=[pl.BlockSpec((1, 1, 4), lambda b: (b, 0, 0)),
                   pl.BlockSpec((1, 1, 4), lambda b: (b, 0, 0))],
        out_shape=[jax.ShapeDtypeStruct((8, 1, 4), jnp.int32),
                   jax.ShapeDtypeStruct((8, 1, 4), jnp.float32)],
    )(infeat)

    idx = idx8.reshape(8, 2, 2)
    vals = vals8.reshape(8, 2, 2)
    idxn = jnp.transpose(idx, (1, 2, 0)).reshape(32)
    provalue = jnp.transpose(vals, (1, 2, 0)).reshape(32)
    py = idxn // _POOL
    px = idxn - py * _POOL
    bn = jnp.arange(32, dtype=jnp.int32) % 8
    pref = jnp.stack([bn, py, px], axis=1).astype(jnp.int32)

    grid_spec = pltpu.PrefetchScalarGridSpec(
        num_scalar_prefetch=1,
        grid=(32,),
        in_specs=[pl.BlockSpec(memory_space=pl.ANY)] * 4,
        out_specs=[
            pl.BlockSpec((1, 2, _K, _K), lambda n, pref: (n, 0, 0, 0)),
            pl.BlockSpec((1, 128, _K, _K), lambda n, pref: (n, 0, 0, 0)),
            pl.BlockSpec((1, 1, _K, _K), lambda n, pref: (n, 0, 0, 0)),
            pl.BlockSpec((1, 1, _K, _K), lambda n, pref: (n, 0, 0, 0)),
        ],
        scratch_shapes=[
            pltpu.MemorySpace.VMEM((2, 40, 256), jnp.float32),
            pltpu.MemorySpace.VMEM((128, 40, 256), jnp.float32),
            pltpu.MemorySpace.VMEM((1, 40, 256), jnp.float32),
            pltpu.MemorySpace.VMEM((1, 40, 256), jnp.float32),
        ] + [pltpu.SemaphoreType.DMA] * 4,
    )
    cls, fda, lt, lps = pl.pallas_call(
        _gather_kernel,
        grid_spec=grid_spec,
        out_shape=[
            jax.ShapeDtypeStruct((32, 2, _K, _K), jnp.float32),
            jax.ShapeDtypeStruct((32, 128, _K, _K), jnp.float32),
            jax.ShapeDtypeStruct((32, 1, _K, _K), jnp.float32),
            jax.ShapeDtypeStruct((32, 1, _K, _K), jnp.float32),
        ],
    )(pref, infeat, labelTpesudo, labelT, FeatureDA)

    return (cls, fda, lt, lps, provalue)


# speedup vs baseline: 19.5048x; 1.4446x over previous
import jax
import jax.numpy as jnp
from jax.experimental import pallas as pl
from jax.experimental.pallas import tpu as pltpu

_POOL = 225
_K = 32
_HALF = 16
_NEG = -1e30


def _score_select_kernel(infeat_ref, idx_ref, val_ref):
    x0 = infeat_ref[0, 0]
    x1 = infeat_ref[0, 1]
    m = jnp.maximum(x0, x1)
    e0 = jnp.exp(x0 - m)
    e1 = jnp.exp(x1 - m)
    inv_s = 1.0 / (e0 + e1)
    p0 = e0 * inv_s
    p1 = e1 * inv_s
    ent = -(p0 * jnp.log(p0 + 1e-5) + p1 * jnp.log(p1 + 1e-5))

    rows = jax.lax.broadcasted_iota(jnp.int32, (256, 256), 0)
    cols = jax.lax.broadcasted_iota(jnp.int32, (256, 256), 1)
    band = ((rows >= cols) & (rows < cols + _K) & (cols < _POOL)).astype(
        jnp.float32)
    valid = (rows < _POOL) & (cols < _POOL)
    flat = rows * _POOL + cols

    idxs = []
    vals = []
    for c in range(2):
        g = (p0 if c == 0 else p1) - 0.1 * ent
        tmp = jax.lax.dot_general(
            g, band, (((1,), (0,)), ((), ())),
            preferred_element_type=jnp.float32,
            precision=jax.lax.Precision.HIGHEST)
        score = jax.lax.dot_general(
            band, tmp, (((0,), (0,)), ((), ())),
            preferred_element_type=jnp.float32,
            precision=jax.lax.Precision.HIGHEST)
        score = jnp.where(valid, score * (1.0 / (_K * _K)), _NEG)
        for kk in range(2):
            v = jnp.max(score)
            idx = jnp.min(jnp.where(score == v, flat, jnp.int32(2**31 - 1)))
            py = idx // _POOL
            px = idx - py * _POOL
            idxs.append(idx)
            vals.append(v)
            in_box = ((rows >= py - _HALF) & (rows < py + _HALF) &
                      (cols >= px - _HALF) & (cols < px + _HALF) & valid)
            score = jnp.where(in_box, 0.0, score)
    idx_ref[...] = jnp.stack(idxs).reshape(1, 1, 4)
    val_ref[...] = jnp.stack(vals).reshape(1, 1, 4)


def _extract(x, dy, px):
    sh_y = jax.lax.rem(40 - dy, 40)
    sh_x = jax.lax.rem(256 - px, 256)
    xr = pltpu.roll(x, sh_y, axis=1)[:, :_K, :]
    xc = pltpu.roll(xr, sh_x, axis=2)[:, :, :_K]
    return xc


def _gather_kernel(pref_ref, infeat_hbm, lps_hbm, lt_hbm, fda_hbm,
                   cls_out, fda_out, lt_out, lps_out,
                   s_if, s_fda, s_lt, s_lps,
                   sems):
    n = pl.program_id(0)
    bufs = (s_if, s_fda, s_lt, s_lps)
    srcs = (infeat_hbm, fda_hbm, lt_hbm, lps_hbm)

    def copies(m, slot):
        b = pref_ref[m, 0]
        py0 = (pref_ref[m, 1] // 8) * 8
        return [
            pltpu.make_async_copy(
                src.at[b, :, pl.ds(py0, 40), :], buf.at[slot], sems.at[slot, j])
            for j, (src, buf) in enumerate(zip(srcs, bufs))
        ]

    @pl.when(n == 0)
    def _():
        for c in copies(0, 0):
            c.start()

    @pl.when(n + 1 < 32)
    def _():
        for c in copies(n + 1, (n + 1) % 2):
            c.start()

    slot = n % 2
    px = pref_ref[n, 2]
    dy = pref_ref[n, 1] - (pref_ref[n, 1] // 8) * 8
    for c, (buf, out) in zip(copies(n, slot),
                             ((s_if, cls_out), (s_fda, fda_out),
                              (s_lt, lt_out), (s_lps, lps_out))):
        c.wait()
        out[0] = _extract(buf[slot], dy, px)


def kernel(infeat, labelTpesudo, labelT, FeatureDA):
    idx8, vals8 = pl.pallas_call(
        _score_select_kernel,
        grid=(8,),
        in_specs=[pl.BlockSpec((1, 2, 256, 256), lambda b: (b, 0, 0, 0))],
        out_specs=[pl.BlockSpec((1, 1, 4), lambda b: (b, 0, 0)),
                   pl.BlockSpec((1, 1, 4), lambda b: (b, 0, 0))],
        out_shape=[jax.ShapeDtypeStruct((8, 1, 4), jnp.int32),
                   jax.ShapeDtypeStruct((8, 1, 4), jnp.float32)],
    )(infeat)

    idx = idx8.reshape(8, 2, 2)
    vals = vals8.reshape(8, 2, 2)
    idxn = jnp.transpose(idx, (1, 2, 0)).reshape(32)
    provalue = jnp.transpose(vals, (1, 2, 0)).reshape(32)
    py = idxn // _POOL
    px = idxn - py * _POOL
    bn = jnp.arange(32, dtype=jnp.int32) % 8
    pref = jnp.stack([bn, py, px], axis=1).astype(jnp.int32)

    grid_spec = pltpu.PrefetchScalarGridSpec(
        num_scalar_prefetch=1,
        grid=(32,),
        in_specs=[pl.BlockSpec(memory_space=pl.ANY)] * 4,
        out_specs=[
            pl.BlockSpec((1, 2, _K, _K), lambda n, pref: (n, 0, 0, 0)),
            pl.BlockSpec((1, 128, _K, _K), lambda n, pref: (n, 0, 0, 0)),
            pl.BlockSpec((1, 1, _K, _K), lambda n, pref: (n, 0, 0, 0)),
            pl.BlockSpec((1, 1, _K, _K), lambda n, pref: (n, 0, 0, 0)),
        ],
        scratch_shapes=[
            pltpu.MemorySpace.VMEM((2, 2, 40, 256), jnp.float32),
            pltpu.MemorySpace.VMEM((2, 128, 40, 256), jnp.float32),
            pltpu.MemorySpace.VMEM((2, 1, 40, 256), jnp.float32),
            pltpu.MemorySpace.VMEM((2, 1, 40, 256), jnp.float32),
            pltpu.SemaphoreType.DMA((2, 4)),
        ],
    )
    cls, fda, lt, lps = pl.pallas_call(
        _gather_kernel,
        grid_spec=grid_spec,
        out_shape=[
            jax.ShapeDtypeStruct((32, 2, _K, _K), jnp.float32),
            jax.ShapeDtypeStruct((32, 128, _K, _K), jnp.float32),
            jax.ShapeDtypeStruct((32, 1, _K, _K), jnp.float32),
            jax.ShapeDtypeStruct((32, 1, _K, _K), jnp.float32),
        ],
    )(pref, infeat, labelTpesudo, labelT, FeatureDA)

    return (cls, fda, lt, lps, provalue)
